# Initial kernel scaffold; baseline (speedup 1.0000x reference)
import jax
import jax.numpy as jnp
from jax.experimental import pallas as pl


def kernel(x, edge_index, batch, u, layers_params, fc_params):
    raise NotImplementedError("write your pallas kernel here")


# speedup vs baseline: 1.7997x; 1.7997x over previous
import functools

import jax
import jax.numpy as jnp
from jax import lax
from jax.experimental import pallas as pl
from jax.experimental.pallas import tpu as pltpu
from jax.experimental.pallas import tpu_sc as plsc

N_NODES = 10000
N_GRAPHS = 8
NC, NS = 2, 16
NW = NC * NS
CH = 128
PER_TILE = 10240
NCHUNK = PER_TILE // CH
E_PAD = NW * PER_TILE
N_PAD = 10240
ZR = N_PAD // NS
BE = 2048
LAT = 64


def _sc_mesh():
    return plsc.VectorSubcoreMesh(core_axis_name="c", subcore_axis_name="s")


_SC_PARAMS = pltpu.CompilerParams(use_tc_tiling_on_sc=False)


def _sc_gather_pair(table, idx_d, idx_s):
    D = table.shape[1]
    out = jax.ShapeDtypeStruct((E_PAD, D), jnp.float32)

    @functools.partial(
        pl.kernel,
        out_type=(out, out),
        mesh=_sc_mesh(),
        compiler_params=_SC_PARAMS,
        scratch_types=[
            pltpu.VMEM((NCHUNK, CH), jnp.int32),
            pltpu.VMEM((NCHUNK, CH), jnp.int32),
            pltpu.VMEM((CH, D), jnp.float32),
            pltpu.VMEM((CH, D), jnp.float32),
        ],
    )
    def k(table_hbm, idxd_hbm, idxs_hbm, outd_hbm, outs_hbm,
          idxd_v, idxs_v, bufd, bufs):
        wid = lax.axis_index("s") * NC + lax.axis_index("c")
        pltpu.sync_copy(idxd_hbm.at[wid], idxd_v)
        pltpu.sync_copy(idxs_hbm.at[wid], idxs_v)
        base = wid * PER_TILE

        def body(j, c):
            off = pl.multiple_of(base + j * CH, CH)
            pltpu.sync_copy(table_hbm.at[idxd_v.at[j]], bufd)
            pltpu.sync_copy(bufd, outd_hbm.at[pl.ds(off, CH)])
            pltpu.sync_copy(table_hbm.at[idxs_v.at[j]], bufs)
            pltpu.sync_copy(bufs, outs_hbm.at[pl.ds(off, CH)])
            return c

        lax.fori_loop(0, NCHUNK, body, 0)

    return k(table, idx_d, idx_s)


def _sc_scatter(msg, idx_d, zeros_init):
    @functools.partial(
        pl.kernel,
        out_type=jax.ShapeDtypeStruct((NC, N_PAD, LAT), jnp.float32),
        mesh=_sc_mesh(),
        compiler_params=_SC_PARAMS,
        scratch_types=[
            pltpu.VMEM((NCHUNK, CH), jnp.int32),
            pltpu.VMEM((CH, LAT), jnp.float32),
            pltpu.VMEM_SHARED((N_PAD, LAT), jnp.float32),
        ],
    )
    def k(msg_hbm, idx_hbm, zero_hbm, out_hbm, idx_v, buf, acc_sh):
        cid = lax.axis_index("c")
        sid = lax.axis_index("s")
        wid = sid * NC + cid
        rows = pl.ds(sid * ZR, ZR)
        pltpu.sync_copy(zero_hbm.at[rows], acc_sh.at[rows])
        pltpu.sync_copy(idx_hbm.at[wid], idx_v)
        plsc.subcore_barrier()
        base = wid * PER_TILE

        def body(j, c):
            off = pl.multiple_of(base + j * CH, CH)
            pltpu.sync_copy(msg_hbm.at[pl.ds(off, CH)], buf)
            pltpu.sync_copy(buf, acc_sh.at[idx_v.at[j]], add=True)
            return c

        lax.fori_loop(0, NCHUNK, body, 0)
        plsc.subcore_barrier()
        pltpu.sync_copy(acc_sh.at[rows], out_hbm.at[cid, rows])

    return k(msg, idx_d, zeros_init)


def _ln(h, g, b):
    m = jnp.mean(h, axis=-1, keepdims=True)
    d = h - m
    v = jnp.mean(d * d, axis=-1, keepdims=True)
    return d * jax.lax.rsqrt(v + 1e-5) * g + b


def _tc_edge_mlp(hd, hs, wdt, wst, w2t, w3t, vrow):
    in_c = hd.shape[1]

    def body(hd_ref, hs_ref, wd_ref, ws_ref, w2_ref, w3_ref, v_ref, out_ref):
        hdv = hd_ref[...]
        hsv = hs_ref[...]
        v = v_ref[...]
        b1, g1, be1 = v[0][None, :], v[1][None, :], v[2][None, :]
        b2, g2, be2 = v[3][None, :], v[4][None, :], v[5][None, :]
        w1c = v[6][None, :]
        b3 = v[7, :LAT][None, :]
        mask3 = (lax.broadcasted_iota(jnp.int32, (1, in_c), 1) < 3
                 ).astype(jnp.float32)
        dif = (hsv - hdv) * mask3
        sq = jnp.sum(dif * dif, axis=1, keepdims=True)
        h1 = (jnp.dot(hdv, wd_ref[...], preferred_element_type=jnp.float32)
              + jnp.dot(hsv, ws_ref[...], preferred_element_type=jnp.float32)
              + sq * w1c + b1)
        a1 = jax.nn.relu(_ln(h1, g1, be1))
        h2 = jnp.dot(a1, w2_ref[...], preferred_element_type=jnp.float32) + b2
        a2 = jax.nn.relu(_ln(h2, g2, be2))
        out_ref[...] = (jnp.dot(a2, w3_ref[...],
                                preferred_element_type=jnp.float32) + b3)

    full = lambda a: pl.BlockSpec(a.shape, lambda i: (0,) * a.ndim)
    return pl.pallas_call(
        body,
        grid=(E_PAD // BE,),
        in_specs=[
            pl.BlockSpec((BE, in_c), lambda i: (i, 0)),
            pl.BlockSpec((BE, in_c), lambda i: (i, 0)),
            full(wdt), full(wst), full(w2t), full(w3t), full(vrow),
        ],
        out_specs=pl.BlockSpec((BE, LAT), lambda i: (i, 0)),
        out_shape=jax.ShapeDtypeStruct((E_PAD, LAT), jnp.float32),
    )(hd, hs, wdt, wst, w2t, w3t, vrow)


def _tc_combine(acc):
    BN = 2000

    def body(a_ref, o_ref):
        o_ref[...] = a_ref[0] + a_ref[1]

    return pl.pallas_call(
        body,
        grid=(N_NODES // BN,),
        in_specs=[pl.BlockSpec((NC, BN, LAT), lambda i: (0, i, 0))],
        out_specs=pl.BlockSpec((BN, LAT), lambda i: (i, 0)),
        out_shape=jax.ShapeDtypeStruct((N_NODES, LAT), jnp.float32),
    )(acc)


def _tc_final(acc, batchf, ones_col, upad, wat, wmt, wxt, wut, w2t, w3t, vrow):
    def body(a_ref, b_ref, one_ref, u_ref, wa_ref, wm_ref, wx_ref, wu_ref,
             w2_ref, w3_ref, v_ref, o_ref):
        h = jax.nn.relu(a_ref[0] + a_ref[1])
        bf = b_ref[...]
        grow = lax.broadcasted_iota(jnp.int32, (1, N_GRAPHS), 1
                                    ).astype(jnp.float32)
        onehot = (bf == grow).astype(jnp.float32)
        cdims = (((0,), (0,)), ((), ()))
        addp = lax.dot_general(onehot, h, cdims,
                               preferred_element_type=jnp.float32)
        cnt = lax.dot_general(onehot, one_ref[...], cdims,
                              preferred_element_type=jnp.float32)
        meanp = addp / jnp.maximum(cnt, 1.0)
        mrows = []
        for g in range(N_GRAPHS):
            mg = bf == jnp.float32(g)
            mrows.append(jnp.max(jnp.where(mg, h, -jnp.inf), axis=0,
                                 keepdims=True))
        maxp = jnp.concatenate(mrows, axis=0)
        v = v_ref[...]
        b1, g1, be1 = v[0][None, :], v[1][None, :], v[2][None, :]
        b2, g2, be2 = v[3][None, :], v[4][None, :], v[5][None, :]
        b3 = v[6, :8][None, :]
        h1 = (jnp.dot(addp, wa_ref[...], preferred_element_type=jnp.float32)
              + jnp.dot(meanp, wm_ref[...], preferred_element_type=jnp.float32)
              + jnp.dot(maxp, wx_ref[...], preferred_element_type=jnp.float32)
              + jnp.dot(u_ref[...], wu_ref[...],
                        preferred_element_type=jnp.float32) + b1)
        a1 = jax.nn.relu(_ln(h1, g1, be1))
        h2 = jnp.dot(a1, w2_ref[...], preferred_element_type=jnp.float32) + b2
        a2 = jax.nn.relu(_ln(h2, g2, be2))
        o_ref[...] = (jnp.dot(a2, w3_ref[...],
                              preferred_element_type=jnp.float32) + b3)

    args = (acc, batchf, ones_col, upad, wat, wmt, wxt, wut, w2t, w3t, vrow)
    full = lambda a: pl.BlockSpec(a.shape, lambda: (0,) * a.ndim)
    return pl.pallas_call(
        body,
        in_specs=[full(a) for a in args],
        out_specs=pl.BlockSpec((8, 8), lambda: (0, 0)),
        out_shape=jax.ShapeDtypeStruct((8, 8), jnp.float32),
    )(*args)


def _prep_edge_weights(p, in_c):
    W1, b1, g1, be1, W2, b2, g2, be2, W3, b3 = p
    wdt = W1[:, :in_c].T
    wst = jnp.concatenate(
        [jnp.zeros((3, W1.shape[0]), jnp.float32),
         W1[:, in_c:2 * in_c - 3].T], axis=0)
    vrow = jnp.stack([b1, g1, be1, b2, g2, be2, W1[:, -1],
                      jnp.pad(b3, (0, W1.shape[0] - LAT))], axis=0)
    return wdt, wst, W2.T, W3.T, vrow


def kernel(x, edge_index, batch, u, layers_params, fc_params):
    src = edge_index[0]
    dst = edge_index[1]
    e = src.shape[0]
    pad = E_PAD - e
    srcp = jnp.concatenate([src, jnp.zeros((pad,), jnp.int32)]
                           ).reshape(NW, NCHUNK, CH)
    dstp = jnp.concatenate([dst, jnp.zeros((pad,), jnp.int32)]
                           ).reshape(NW, NCHUNK, CH)
    dsts = jnp.concatenate([dst, jnp.full((pad,), N_NODES, jnp.int32)]
                           ).reshape(NW, NCHUNK, CH)
    zeros_init = jnp.zeros((N_PAD, LAT), jnp.float32)

    h = x
    acc = None
    for li, p in enumerate(layers_params):
        in_c = h.shape[1]
        wdt, wst, w2t, w3t, vrow = _prep_edge_weights(p, in_c)
        hd, hs = _sc_gather_pair(h, dstp, srcp)
        msg = _tc_edge_mlp(hd, hs, wdt, wst, w2t, w3t, vrow)
        acc = _sc_scatter(msg, dsts, zeros_init)
        if li < len(layers_params) - 1:
            h = _tc_combine(acc)

    W1, b1, g1, be1, W2, b2, g2, be2, W3, b3 = fc_params
    batchf = jnp.pad(batch.astype(jnp.float32)[:, None],
                     ((0, N_PAD - N_NODES), (0, 0)), constant_values=8.0)
    ones_col = jnp.ones((N_PAD, 1), jnp.float32)
    upad = jnp.pad(u, ((0, 0), (0, 6)))
    wat = W1[:, :LAT].T
    wmt = W1[:, LAT:2 * LAT].T
    wxt = W1[:, 2 * LAT:3 * LAT].T
    wut = jnp.pad(W1[:, 3 * LAT:].T, ((0, 6), (0, 0)))
    w3t = jnp.pad(W3.T, ((0, 0), (0, 8 - W3.shape[0])))
    vrow = jnp.stack([b1, g1, be1, b2, g2, be2,
                      jnp.pad(b3, (0, LAT - b3.shape[0]))], axis=0)
    out8 = _tc_final(acc, batchf, ones_col, upad, wat, wmt, wxt, wut,
                     W2.T, w3t, vrow)
    return out8[:, :W3.shape[0]]


# speedup vs baseline: 2.0649x; 1.1473x over previous
import functools

import jax
import jax.numpy as jnp
from jax import lax
from jax.experimental import pallas as pl
from jax.experimental.pallas import tpu as pltpu
from jax.experimental.pallas import tpu_sc as plsc

N_NODES = 10000
N_GRAPHS = 8
NC, NS = 2, 16
NW = NC * NS
CH = 128
PER_TILE = 10240
NCHUNK = PER_TILE // CH
E_PAD = NW * PER_TILE
N_PAD = 10240
ZR = N_PAD // NS
BE = 2048
LAT = 64


def _sc_mesh():
    return plsc.VectorSubcoreMesh(core_axis_name="c", subcore_axis_name="s")


_SC_PARAMS = pltpu.CompilerParams(use_tc_tiling_on_sc=False)


def _sc_gather_pair(table, idx_d, idx_s):
    D = table.shape[1]
    out = jax.ShapeDtypeStruct((E_PAD, D), jnp.float32)

    NB = 2

    @functools.partial(
        pl.kernel,
        out_type=(out, out),
        mesh=_sc_mesh(),
        compiler_params=_SC_PARAMS,
        scratch_types=[
            pltpu.VMEM((NCHUNK, CH), jnp.int32),
            pltpu.VMEM((NCHUNK, CH), jnp.int32),
            [pltpu.VMEM((CH, D), jnp.float32) for _ in range(NB)],
            [pltpu.VMEM((CH, D), jnp.float32) for _ in range(NB)],
            [pltpu.SemaphoreType.DMA for _ in range(4 * NB)],
        ],
    )
    def k(table_hbm, idxd_hbm, idxs_hbm, outd_hbm, outs_hbm,
          idxd_v, idxs_v, bufd, bufs, sems):
        gd, gs = sems[:NB], sems[NB:2 * NB]
        wd, ws = sems[2 * NB:3 * NB], sems[3 * NB:]
        wid = lax.axis_index("s") * NC + lax.axis_index("c")
        pltpu.sync_copy(idxd_hbm.at[wid], idxd_v)
        pltpu.sync_copy(idxs_hbm.at[wid], idxs_v)
        base = wid * PER_TILE

        for b in range(NB):
            pltpu.async_copy(table_hbm.at[idxd_v.at[b]], bufd[b], gd[b])
            pltpu.async_copy(table_hbm.at[idxs_v.at[b]], bufs[b], gs[b])

        def body(g, c):
            for b in range(NB):
                j = g + b
                off = pl.multiple_of(base + j * CH, CH)
                pltpu.make_async_copy(table_hbm.at[idxd_v.at[j]], bufd[b],
                                      gd[b]).wait()
                pltpu.async_copy(bufd[b], outd_hbm.at[pl.ds(off, CH)], wd[b])
                pltpu.make_async_copy(table_hbm.at[idxs_v.at[j]], bufs[b],
                                      gs[b]).wait()
                pltpu.async_copy(bufs[b], outs_hbm.at[pl.ds(off, CH)], ws[b])
                nxt = j + NB

                @pl.when(nxt < NCHUNK)
                def _():
                    pltpu.make_async_copy(bufd[b], outd_hbm.at[pl.ds(off, CH)],
                                          wd[b]).wait()
                    pltpu.async_copy(table_hbm.at[idxd_v.at[nxt]], bufd[b],
                                     gd[b])
                    pltpu.make_async_copy(bufs[b], outs_hbm.at[pl.ds(off, CH)],
                                          ws[b]).wait()
                    pltpu.async_copy(table_hbm.at[idxs_v.at[nxt]], bufs[b],
                                     gs[b])

            return c

        lax.fori_loop(0, NCHUNK // NB, lambda i, c: body(i * NB, c), 0)
        for b in range(NB):
            off = pl.multiple_of(base + (NCHUNK - NB + b) * CH, CH)
            pltpu.make_async_copy(bufd[b], outd_hbm.at[pl.ds(off, CH)],
                                  wd[b]).wait()
            pltpu.make_async_copy(bufs[b], outs_hbm.at[pl.ds(off, CH)],
                                  ws[b]).wait()

    return k(table, idx_d, idx_s)


def _sc_scatter(msg, idx_d, zeros_init):
    RB = 512
    Q = RB // CH
    NR = PER_TILE // RB

    @functools.partial(
        pl.kernel,
        out_type=jax.ShapeDtypeStruct((NC, N_PAD, LAT), jnp.float32),
        mesh=_sc_mesh(),
        compiler_params=_SC_PARAMS,
        scratch_types=[
            pltpu.VMEM((NCHUNK, CH), jnp.int32),
            [pltpu.VMEM((RB, LAT), jnp.float32) for _ in range(2)],
            pltpu.VMEM_SHARED((N_PAD, LAT), jnp.float32),
            [pltpu.SemaphoreType.DMA for _ in range(4)],
        ],
    )
    def k(msg_hbm, idx_hbm, zero_hbm, out_hbm, idx_v, bufm, acc_sh, sems):
        rm, sa = sems[:2], sems[2:]
        cid = lax.axis_index("c")
        sid = lax.axis_index("s")
        wid = sid * NC + cid
        rows = pl.ds(sid * ZR, ZR)
        pltpu.sync_copy(zero_hbm.at[rows], acc_sh.at[rows])
        pltpu.sync_copy(idx_hbm.at[wid], idx_v)
        plsc.subcore_barrier()
        base = wid * PER_TILE

        def rd(r, b):
            off = pl.multiple_of(base + r * RB, RB)
            return pltpu.make_async_copy(msg_hbm.at[pl.ds(off, RB)], bufm[b],
                                         rm[b])

        def add_start(r, b, q):
            pltpu.async_copy(bufm[b].at[pl.ds(q * CH, CH)],
                             acc_sh.at[idx_v.at[r * Q + q]], sa[b], add=True)

        def add_wait(r, b, q):
            pltpu.make_async_copy(bufm[b].at[pl.ds(q * CH, CH)],
                                  acc_sh.at[idx_v.at[r * Q + q]], sa[b]).wait()

        rd(0, 0).start()
        rd(1, 1).start()

        def body(g, c):
            for b in range(2):
                r = g + b
                rd(r, b).wait()
                for q in range(Q):
                    add_start(r, b, q)

                @pl.when(r + 2 < NR)
                def _():
                    for q in range(Q):
                        add_wait(r, b, q)
                    rd(r + 2, b).start()

            return c

        lax.fori_loop(0, NR // 2, lambda i, c: body(i * 2, c), 0)
        for b in range(2):
            r = NR - 2 + b
            for q in range(Q):
                add_wait(r, b, q)
        plsc.subcore_barrier()
        pltpu.sync_copy(acc_sh.at[rows], out_hbm.at[cid, rows])

    return k(msg, idx_d, zeros_init)


def _ln(h, g, b):
    m = jnp.mean(h, axis=-1, keepdims=True)
    d = h - m
    v = jnp.mean(d * d, axis=-1, keepdims=True)
    return d * jax.lax.rsqrt(v + 1e-5) * g + b


def _tc_edge_mlp(hd, hs, wdt, wst, w2t, w3t, vrow):
    in_c = hd.shape[1]

    def body(hd_ref, hs_ref, wd_ref, ws_ref, w2_ref, w3_ref, v_ref, out_ref):
        hdv = hd_ref[...]
        hsv = hs_ref[...]
        v = v_ref[...]
        b1, g1, be1 = v[0][None, :], v[1][None, :], v[2][None, :]
        b2, g2, be2 = v[3][None, :], v[4][None, :], v[5][None, :]
        w1c = v[6][None, :]
        b3 = v[7, :LAT][None, :]
        mask3 = (lax.broadcasted_iota(jnp.int32, (1, in_c), 1) < 3
                 ).astype(jnp.float32)
        dif = (hsv - hdv) * mask3
        sq = jnp.sum(dif * dif, axis=1, keepdims=True)
        h1 = (jnp.dot(hdv, wd_ref[...], preferred_element_type=jnp.float32)
              + jnp.dot(hsv, ws_ref[...], preferred_element_type=jnp.float32)
              + sq * w1c + b1)
        a1 = jax.nn.relu(_ln(h1, g1, be1))
        h2 = jnp.dot(a1, w2_ref[...], preferred_element_type=jnp.float32) + b2
        a2 = jax.nn.relu(_ln(h2, g2, be2))
        out_ref[...] = (jnp.dot(a2, w3_ref[...],
                                preferred_element_type=jnp.float32) + b3)

    full = lambda a: pl.BlockSpec(a.shape, lambda i: (0,) * a.ndim)
    return pl.pallas_call(
        body,
        grid=(E_PAD // BE,),
        in_specs=[
            pl.BlockSpec((BE, in_c), lambda i: (i, 0)),
            pl.BlockSpec((BE, in_c), lambda i: (i, 0)),
            full(wdt), full(wst), full(w2t), full(w3t), full(vrow),
        ],
        out_specs=pl.BlockSpec((BE, LAT), lambda i: (i, 0)),
        out_shape=jax.ShapeDtypeStruct((E_PAD, LAT), jnp.float32),
    )(hd, hs, wdt, wst, w2t, w3t, vrow)


def _tc_combine(acc):
    BN = 2000

    def body(a_ref, o_ref):
        o_ref[...] = a_ref[0] + a_ref[1]

    return pl.pallas_call(
        body,
        grid=(N_NODES // BN,),
        in_specs=[pl.BlockSpec((NC, BN, LAT), lambda i: (0, i, 0))],
        out_specs=pl.BlockSpec((BN, LAT), lambda i: (i, 0)),
        out_shape=jax.ShapeDtypeStruct((N_NODES, LAT), jnp.float32),
    )(acc)


def _tc_final(acc, batchf, ones_col, upad, wat, wmt, wxt, wut, w2t, w3t, vrow):
    def body(a_ref, b_ref, one_ref, u_ref, wa_ref, wm_ref, wx_ref, wu_ref,
             w2_ref, w3_ref, v_ref, o_ref):
        h = jax.nn.relu(a_ref[0] + a_ref[1])
        bf = b_ref[...]
        grow = lax.broadcasted_iota(jnp.int32, (1, N_GRAPHS), 1
                                    ).astype(jnp.float32)
        onehot = (bf == grow).astype(jnp.float32)
        cdims = (((0,), (0,)), ((), ()))
        addp = lax.dot_general(onehot, h, cdims,
                               preferred_element_type=jnp.float32)
        cnt = lax.dot_general(onehot, one_ref[...], cdims,
                              preferred_element_type=jnp.float32)
        meanp = addp / jnp.maximum(cnt, 1.0)
        mrows = []
        for g in range(N_GRAPHS):
            mg = bf == jnp.float32(g)
            mrows.append(jnp.max(jnp.where(mg, h, -jnp.inf), axis=0,
                                 keepdims=True))
        maxp = jnp.concatenate(mrows, axis=0)
        v = v_ref[...]
        b1, g1, be1 = v[0][None, :], v[1][None, :], v[2][None, :]
        b2, g2, be2 = v[3][None, :], v[4][None, :], v[5][None, :]
        b3 = v[6, :8][None, :]
        h1 = (jnp.dot(addp, wa_ref[...], preferred_element_type=jnp.float32)
              + jnp.dot(meanp, wm_ref[...], preferred_element_type=jnp.float32)
              + jnp.dot(maxp, wx_ref[...], preferred_element_type=jnp.float32)
              + jnp.dot(u_ref[...], wu_ref[...],
                        preferred_element_type=jnp.float32) + b1)
        a1 = jax.nn.relu(_ln(h1, g1, be1))
        h2 = jnp.dot(a1, w2_ref[...], preferred_element_type=jnp.float32) + b2
        a2 = jax.nn.relu(_ln(h2, g2, be2))
        o_ref[...] = (jnp.dot(a2, w3_ref[...],
                              preferred_element_type=jnp.float32) + b3)

    args = (acc, batchf, ones_col, upad, wat, wmt, wxt, wut, w2t, w3t, vrow)
    full = lambda a: pl.BlockSpec(a.shape, lambda: (0,) * a.ndim)
    return pl.pallas_call(
        body,
        in_specs=[full(a) for a in args],
        out_specs=pl.BlockSpec((8, 8), lambda: (0, 0)),
        out_shape=jax.ShapeDtypeStruct((8, 8), jnp.float32),
    )(*args)


def _prep_edge_weights(p, in_c):
    W1, b1, g1, be1, W2, b2, g2, be2, W3, b3 = p
    wdt = W1[:, :in_c].T
    wst = jnp.concatenate(
        [jnp.zeros((3, W1.shape[0]), jnp.float32),
         W1[:, in_c:2 * in_c - 3].T], axis=0)
    vrow = jnp.stack([b1, g1, be1, b2, g2, be2, W1[:, -1],
                      jnp.pad(b3, (0, W1.shape[0] - LAT))], axis=0)
    return wdt, wst, W2.T, W3.T, vrow


def kernel(x, edge_index, batch, u, layers_params, fc_params):
    src = edge_index[0]
    dst = edge_index[1]
    e = src.shape[0]
    pad = E_PAD - e
    srcp = jnp.concatenate([src, jnp.zeros((pad,), jnp.int32)]
                           ).reshape(NW, NCHUNK, CH)
    dstp = jnp.concatenate([dst, jnp.zeros((pad,), jnp.int32)]
                           ).reshape(NW, NCHUNK, CH)
    dsts = jnp.concatenate([dst, jnp.full((pad,), N_NODES, jnp.int32)]
                           ).reshape(NW, NCHUNK, CH)
    zeros_init = jnp.zeros((N_PAD, LAT), jnp.float32)

    h = x
    acc = None
    for li, p in enumerate(layers_params):
        in_c = h.shape[1]
        wdt, wst, w2t, w3t, vrow = _prep_edge_weights(p, in_c)
        hd, hs = _sc_gather_pair(h, dstp, srcp)
        msg = _tc_edge_mlp(hd, hs, wdt, wst, w2t, w3t, vrow)
        acc = _sc_scatter(msg, dsts, zeros_init)
        if li < len(layers_params) - 1:
            h = _tc_combine(acc)

    W1, b1, g1, be1, W2, b2, g2, be2, W3, b3 = fc_params
    batchf = jnp.pad(batch.astype(jnp.float32)[:, None],
                     ((0, N_PAD - N_NODES), (0, 0)), constant_values=8.0)
    ones_col = jnp.ones((N_PAD, 1), jnp.float32)
    upad = jnp.pad(u, ((0, 0), (0, 6)))
    wat = W1[:, :LAT].T
    wmt = W1[:, LAT:2 * LAT].T
    wxt = W1[:, 2 * LAT:3 * LAT].T
    wut = jnp.pad(W1[:, 3 * LAT:].T, ((0, 6), (0, 0)))
    w3t = jnp.pad(W3.T, ((0, 0), (0, 8 - W3.shape[0])))
    vrow = jnp.stack([b1, g1, be1, b2, g2, be2,
                      jnp.pad(b3, (0, LAT - b3.shape[0]))], axis=0)
    out8 = _tc_final(acc, batchf, ones_col, upad, wat, wmt, wxt, wut,
                     W2.T, w3t, vrow)
    return out8[:, :W3.shape[0]]
